# Initial kernel scaffold; baseline (speedup 1.0000x reference)
import functools

import jax
import jax.numpy as jnp
from jax import lax
from jax.experimental import pallas as pl
from jax.experimental.pallas import tpu as pltpu
from jax.experimental.pallas import tpu_sc as plsc

VOCAB = 100000
EMBED = 64
B = 4096
T = 200
BFLAT = B * T


@functools.cache
def _build(num_cores: int, num_subcores: int):
    nw = num_cores * num_subcores
    b_per_w = BFLAT // nw
    ch = 128
    n_chunks = b_per_w // ch

    mesh = plsc.VectorSubcoreMesh(core_axis_name="c", subcore_axis_name="s")

    @functools.partial(
        pl.kernel,
        out_type=jax.ShapeDtypeStruct((BFLAT, EMBED), jnp.float32),
        mesh=mesh,
        scratch_types=[
            pltpu.VMEM((n_chunks, ch), jnp.int32),
            pltpu.VMEM((ch, EMBED), jnp.float32),
            pltpu.SemaphoreType.DMA,
        ],
    )
    def gather_kernel(tok_hbm, table_hbm, out_hbm, idx_v, rows_v, sem):
        wid = lax.axis_index("s") * num_cores + lax.axis_index("c")
        base = wid * b_per_w
        pltpu.sync_copy(tok_hbm.at[wid], idx_v)

        @pl.loop(0, n_chunks)
        def _(c):
            pltpu.async_copy(table_hbm.at[idx_v.at[c]], rows_v, sem).wait()
            pltpu.sync_copy(rows_v, out_hbm.at[pl.ds(base + c * ch, ch)])

    return gather_kernel, nw, n_chunks, ch


def kernel(token_ids, table):
    info = plsc.get_sparse_core_info()
    fn, nw, n_chunks, ch = _build(info.num_cores, info.num_subcores)
    tok = token_ids.astype(jnp.int32).reshape(nw, n_chunks, ch)
    out = fn(tok, table)
    return out.reshape(B, T, EMBED)


# speedup vs baseline: 3.5476x; 3.5476x over previous
import functools

import jax
import jax.numpy as jnp
from jax import lax
from jax.experimental import pallas as pl
from jax.experimental.pallas import tpu as pltpu
from jax.experimental.pallas import tpu_sc as plsc

VOCAB = 100000
EMBED = 64
B = 4096
T = 200
BFLAT = B * T


@functools.cache
def _build(num_cores: int, num_subcores: int):
    nw = num_cores * num_subcores
    b_per_w = BFLAT // nw
    ch = 128
    n_chunks = b_per_w // ch

    mesh = plsc.VectorSubcoreMesh(core_axis_name="c", subcore_axis_name="s")

    @functools.partial(
        pl.kernel,
        out_type=jax.ShapeDtypeStruct((BFLAT, EMBED), jnp.float32),
        mesh=mesh,
        scratch_types=[
            pltpu.VMEM((n_chunks, ch), jnp.int32),
            pltpu.VMEM((ch, EMBED), jnp.float32),
            pltpu.SemaphoreType.DMA,
        ],
        compiler_params=pltpu.CompilerParams(use_tc_tiling_on_sc=False),
    )
    def gather_kernel(tok_hbm, table_hbm, out_hbm, idx_v, rows_v, sem):
        wid = lax.axis_index("s") * num_cores + lax.axis_index("c")
        base = wid * b_per_w
        pltpu.sync_copy(tok_hbm.at[wid], idx_v)

        @pl.loop(0, n_chunks)
        def _(c):
            pltpu.async_copy(table_hbm.at[idx_v.at[c]], rows_v, sem).wait()
            pltpu.sync_copy(rows_v, out_hbm.at[pl.ds(base + c * ch, ch)])

    return gather_kernel, nw, n_chunks, ch


def kernel(token_ids, table):
    info = plsc.get_sparse_core_info()
    fn, nw, n_chunks, ch = _build(info.num_cores, info.num_subcores)
    tok = token_ids.astype(jnp.int32).reshape(nw, n_chunks, ch)
    out = fn(tok, table)
    return out.reshape(B, T, EMBED)


# speedup vs baseline: 4.2623x; 1.2015x over previous
import functools

import jax
import jax.numpy as jnp
from jax import lax
from jax.experimental import pallas as pl
from jax.experimental.pallas import tpu as pltpu
from jax.experimental.pallas import tpu_sc as plsc

VOCAB = 100000
EMBED = 64
B = 4096
T = 200
BFLAT = B * T


@functools.cache
def _build(num_cores: int, num_subcores: int):
    nw = num_cores * num_subcores
    b_per_w = BFLAT // nw
    ch = 128
    n_chunks = b_per_w // ch
    k = 5
    rows = ch * k
    g_total = n_chunks // k

    mesh = plsc.VectorSubcoreMesh(core_axis_name="c", subcore_axis_name="s")

    @functools.partial(
        pl.kernel,
        out_type=jax.ShapeDtypeStruct((BFLAT, EMBED), jnp.float32),
        mesh=mesh,
        scratch_types=[
            pltpu.VMEM((n_chunks, ch), jnp.int32),
            pltpu.VMEM((rows, EMBED), jnp.float32),
            pltpu.VMEM((rows, EMBED), jnp.float32),
            pltpu.SemaphoreType.DMA,
            pltpu.SemaphoreType.DMA,
        ],
        compiler_params=pltpu.CompilerParams(use_tc_tiling_on_sc=False),
    )
    def gather_kernel(tok_hbm, table_hbm, out_hbm, idx_v, rows0, rows1, sem0, sem1):
        wid = lax.axis_index("s") * num_cores + lax.axis_index("c")
        base = wid * b_per_w
        pltpu.sync_copy(tok_hbm.at[wid], idx_v)

        def fire(g, buf, sem):
            for j in range(k):
                pltpu.async_copy(
                    table_hbm.at[idx_v.at[g * k + j]],
                    buf.at[pl.ds(j * ch, ch)],
                    sem,
                )

        def drain(buf, sem):
            for j in range(k):
                pltpu.make_async_copy(
                    table_hbm.at[idx_v.at[0]],
                    buf.at[pl.ds(j * ch, ch)],
                    sem,
                ).wait()

        fire(0, rows0, sem0)

        @pl.loop(0, g_total, step=2)
        def _(g):
            fire(g + 1, rows1, sem1)
            drain(rows0, sem0)
            pltpu.sync_copy(rows0, out_hbm.at[pl.ds(base + g * rows, rows)])

            @pl.when(g + 2 < g_total)
            def _():
                fire(g + 2, rows0, sem0)

            drain(rows1, sem1)
            pltpu.sync_copy(rows1, out_hbm.at[pl.ds(base + (g + 1) * rows, rows)])

    return gather_kernel, nw, n_chunks, ch


def kernel(token_ids, table):
    info = plsc.get_sparse_core_info()
    fn, nw, n_chunks, ch = _build(info.num_cores, info.num_subcores)
    tok = token_ids.astype(jnp.int32).reshape(nw, n_chunks, ch)
    out = fn(tok, table)
    return out.reshape(B, T, EMBED)
